# baseline (speedup 1.0000x reference)
import functools

import jax
import jax.numpy as jnp
from jax import lax
from jax.experimental import pallas as pl
from jax.experimental.pallas import tpu as pltpu
from jax.experimental.pallas import tpu_sc as plsc

_NC = 2
_NS = 16
_NW = _NC * _NS

_CHUNK = 96


def _segment_sum_sc(h, src, dst):
    N, D = h.shape
    E = src.shape[0]
    per_w = -(-E // (_NW * _CHUNK)) * _CHUNK
    n_chunks = per_w // _CHUNK
    rows_per_tile = -(-N // (_NS * 8)) * 8
    Np = rows_per_tile * _NS

    pad = per_w * _NW - E
    if pad:
        pad_rows = Np - N
        src = jnp.concatenate([src, jnp.zeros((pad,), jnp.int32)])
        dst = jnp.concatenate(
            [dst, N + (jnp.arange(pad, dtype=jnp.int32) % pad_rows)])
    dst3 = dst.reshape(_NW, n_chunks, _CHUNK)

    mesh = plsc.VectorSubcoreMesh(core_axis_name="c", subcore_axis_name="s")

    @functools.partial(
        pl.kernel,
        out_type=jax.ShapeDtypeStruct((_NC, Np, D), jnp.float32),
        mesh=mesh,
        scratch_types=[
            pltpu.VMEM((per_w,), jnp.int32),
            pltpu.VMEM((n_chunks, _CHUNK), jnp.int32),
            pltpu.VMEM((_CHUNK, D), jnp.float32),
            pltpu.VMEM((_CHUNK, D), jnp.float32),
            pltpu.VMEM_SHARED((Np, D), jnp.float32),
            pltpu.SemaphoreType.DMA,
            pltpu.SemaphoreType.DMA,
        ],
    )
    def seg_sum(h_hbm, src_hbm, dst_hbm, out_hbm, sidx_v, didx_v, rows_a,
                rows_b, acc_sh, sem_a, sem_b):
        c = lax.axis_index("c")
        s = lax.axis_index("s")
        wid = s * _NC + c

        pltpu.async_copy(src_hbm.at[pl.ds(wid * per_w, per_w)], sidx_v, sem_a)
        pltpu.async_copy(dst_hbm.at[wid], didx_v, sem_b)

        zeros16 = jnp.zeros((16,), jnp.float32)

        def zfill(i, _):
            r = i // (D // 16)
            col = (i % (D // 16)) * 16
            rows_a[r, pl.ds(col, 16)] = zeros16
            return 0

        lax.fori_loop(0, _CHUNK * (D // 16), zfill, 0)
        nfull = rows_per_tile // _CHUNK
        rem = rows_per_tile % _CHUNK

        def zcopy(k, _):
            pltpu.sync_copy(rows_a, acc_sh.at[pl.ds(s * rows_per_tile + k * _CHUNK, _CHUNK)])
            return 0

        lax.fori_loop(0, nfull, zcopy, 0)
        if rem:
            pltpu.sync_copy(
                rows_a.at[pl.ds(0, rem)],
                acc_sh.at[pl.ds(s * rows_per_tile + nfull * _CHUNK, rem)])
        pltpu.make_async_copy(src_hbm.at[pl.ds(wid * per_w, per_w)], sidx_v,
                              sem_a).wait()
        pltpu.make_async_copy(dst_hbm.at[wid], didx_v, sem_b).wait()
        plsc.subcore_barrier()

        def start_gather(j, rows_v, sem):
            pltpu.async_copy(h_hbm.at[sidx_v.at[pl.ds(j * _CHUNK, _CHUNK)]], rows_v, sem)

        def wait_gather(j, rows_v, sem):
            pltpu.make_async_copy(h_hbm.at[sidx_v.at[pl.ds(j * _CHUNK, _CHUNK)]], rows_v, sem).wait()

        start_gather(0, rows_a, sem_a)
        start_gather(1, rows_b, sem_b)

        def scatter(j, rows_v):
            pltpu.sync_copy(rows_v, acc_sh.at[didx_v.at[j]], add=True)

        def pair(k, _):
            j = 2 * k
            wait_gather(j, rows_a, sem_a)
            scatter(j, rows_a)
            start_gather(j + 2, rows_a, sem_a)
            wait_gather(j + 1, rows_b, sem_b)
            scatter(j + 1, rows_b)
            start_gather(j + 3, rows_b, sem_b)
            return 0

        if n_chunks % 2 == 0:
            lax.fori_loop(0, n_chunks // 2 - 1, pair, 0)
            wait_gather(n_chunks - 2, rows_a, sem_a)
            scatter(n_chunks - 2, rows_a)
            wait_gather(n_chunks - 1, rows_b, sem_b)
            scatter(n_chunks - 1, rows_b)
        else:
            lax.fori_loop(0, (n_chunks - 3) // 2, pair, 0)
            wait_gather(n_chunks - 3, rows_a, sem_a)
            scatter(n_chunks - 3, rows_a)
            start_gather(n_chunks - 1, rows_a, sem_a)
            wait_gather(n_chunks - 2, rows_b, sem_b)
            scatter(n_chunks - 2, rows_b)
            wait_gather(n_chunks - 1, rows_a, sem_a)
            scatter(n_chunks - 1, rows_a)
        plsc.subcore_barrier()

        r0 = s * rows_per_tile
        pltpu.sync_copy(acc_sh.at[pl.ds(r0, rows_per_tile)],
                        out_hbm.at[c, pl.ds(r0, rows_per_tile)])

    return seg_sum(h, src, dst3)


_ROWS_BLK = 2000


def _mlp_body(x_ref, agg_ref, Wa_ref, ba_ref, Wb_ref, bb_ref, o_ref):
    z = x_ref[...] + agg_ref[0] + agg_ref[1]
    h1 = jnp.dot(z, Wa_ref[...], preferred_element_type=jnp.float32)
    h1 = jnp.maximum(h1 + ba_ref[...], 0.0)
    h2 = jnp.dot(h1, Wb_ref[...], preferred_element_type=jnp.float32)
    o_ref[...] = jnp.maximum(h2 + bb_ref[...], 0.0)


def _mlp(x, agg, Wa, ba, Wb, bb):
    N, D = x.shape
    H = Wa.shape[1]
    grid = (N // _ROWS_BLK,)
    return pl.pallas_call(
        _mlp_body,
        grid=grid,
        in_specs=[
            pl.BlockSpec((_ROWS_BLK, D), lambda i: (i, 0)),
            pl.BlockSpec((_NC, _ROWS_BLK, D), lambda i: (0, i, 0)),
            pl.BlockSpec((D, H), lambda i: (0, 0)),
            pl.BlockSpec((1, H), lambda i: (0, 0)),
            pl.BlockSpec((H, H), lambda i: (0, 0)),
            pl.BlockSpec((1, H), lambda i: (0, 0)),
        ],
        out_specs=pl.BlockSpec((_ROWS_BLK, H), lambda i: (i, 0)),
        out_shape=jax.ShapeDtypeStruct((N, H), jnp.float32),
    )(x, agg, Wa, ba, Wb, bb)


def kernel(x, edge_index, W1, b1, W2, b2, W3, b3, W4, b4):
    src = edge_index[0]
    dst = edge_index[1]
    b1r = b1.reshape(1, -1)
    b2r = b2.reshape(1, -1)
    b3r = b3.reshape(1, -1)
    b4r = b4.reshape(1, -1)
    agg1 = _segment_sum_sc(x, src, dst)
    h = _mlp(x, agg1, W1, b1r, W2, b2r)
    agg2 = _segment_sum_sc(h, src, dst)
    out = _mlp(h, agg2, W3, b3r, W4, b4r)
    return out


# speedup vs baseline: 1.8793x; 1.8793x over previous
import functools

import jax
import jax.numpy as jnp
from jax import lax
from jax.experimental import pallas as pl
from jax.experimental.pallas import tpu as pltpu
from jax.experimental.pallas import tpu_sc as plsc

_NC = 2
_NS = 16
_NW = _NC * _NS

_CHUNK = 96


def _segment_sum_sc(h, src, dst):
    N, D = h.shape
    E = src.shape[0]
    per_w = -(-E // (_NW * _CHUNK)) * _CHUNK
    n_chunks = per_w // _CHUNK
    rows_per_tile = -(-N // (_NS * 8)) * 8
    Np = rows_per_tile * _NS

    pad = per_w * _NW - E
    if pad:
        pad_rows = Np - N
        src = jnp.concatenate(
            [src, jnp.arange(pad, dtype=jnp.int32) * (N // pad)])
        dst = jnp.concatenate(
            [dst, N + (jnp.arange(pad, dtype=jnp.int32) % pad_rows)])
    dst3 = dst.reshape(_NW, n_chunks, _CHUNK)

    mesh = plsc.VectorSubcoreMesh(core_axis_name="c", subcore_axis_name="s")

    @functools.partial(
        pl.kernel,
        out_type=jax.ShapeDtypeStruct((_NC, Np, D), jnp.float32),
        mesh=mesh,
        scratch_types=[
            pltpu.VMEM((per_w,), jnp.int32),
            pltpu.VMEM((n_chunks, _CHUNK), jnp.int32),
            pltpu.VMEM((_CHUNK, D), jnp.float32),
            pltpu.VMEM((_CHUNK, D), jnp.float32),
            pltpu.VMEM_SHARED((Np, D), jnp.float32),
            pltpu.SemaphoreType.DMA,
            pltpu.SemaphoreType.DMA,
        ],
    )
    def seg_sum(h_hbm, src_hbm, dst_hbm, out_hbm, sidx_v, didx_v, rows_a,
                rows_b, acc_sh, sem_a, sem_b):
        c = lax.axis_index("c")
        s = lax.axis_index("s")
        wid = s * _NC + c

        pltpu.async_copy(src_hbm.at[pl.ds(wid * per_w, per_w)], sidx_v, sem_a)
        pltpu.async_copy(dst_hbm.at[wid], didx_v, sem_b)

        zeros16 = jnp.zeros((16,), jnp.float32)

        def zfill(i, _):
            r = i // (D // 16)
            col = (i % (D // 16)) * 16
            rows_a[r, pl.ds(col, 16)] = zeros16
            return 0

        lax.fori_loop(0, _CHUNK * (D // 16), zfill, 0)
        nfull = rows_per_tile // _CHUNK
        rem = rows_per_tile % _CHUNK

        def zcopy(k, _):
            pltpu.sync_copy(rows_a, acc_sh.at[pl.ds(s * rows_per_tile + k * _CHUNK, _CHUNK)])
            return 0

        lax.fori_loop(0, nfull, zcopy, 0)
        if rem:
            pltpu.sync_copy(
                rows_a.at[pl.ds(0, rem)],
                acc_sh.at[pl.ds(s * rows_per_tile + nfull * _CHUNK, rem)])
        pltpu.make_async_copy(src_hbm.at[pl.ds(wid * per_w, per_w)], sidx_v,
                              sem_a).wait()
        pltpu.make_async_copy(dst_hbm.at[wid], didx_v, sem_b).wait()
        plsc.subcore_barrier()

        def start_gather(j, rows_v, sem):
            pltpu.async_copy(h_hbm.at[sidx_v.at[pl.ds(j * _CHUNK, _CHUNK)]], rows_v, sem)

        def wait_gather(j, rows_v, sem):
            pltpu.make_async_copy(h_hbm.at[sidx_v.at[pl.ds(j * _CHUNK, _CHUNK)]], rows_v, sem).wait()

        start_gather(0, rows_a, sem_a)
        start_gather(1, rows_b, sem_b)

        def scatter(j, rows_v):
            pltpu.sync_copy(rows_v, acc_sh.at[didx_v.at[j]], add=True)

        def pair(k, _):
            j = 2 * k
            wait_gather(j, rows_a, sem_a)
            scatter(j, rows_a)
            start_gather(j + 2, rows_a, sem_a)
            wait_gather(j + 1, rows_b, sem_b)
            scatter(j + 1, rows_b)
            start_gather(j + 3, rows_b, sem_b)
            return 0

        if n_chunks % 2 == 0:
            lax.fori_loop(0, n_chunks // 2 - 1, pair, 0)
            wait_gather(n_chunks - 2, rows_a, sem_a)
            scatter(n_chunks - 2, rows_a)
            wait_gather(n_chunks - 1, rows_b, sem_b)
            scatter(n_chunks - 1, rows_b)
        else:
            lax.fori_loop(0, (n_chunks - 3) // 2, pair, 0)
            wait_gather(n_chunks - 3, rows_a, sem_a)
            scatter(n_chunks - 3, rows_a)
            start_gather(n_chunks - 1, rows_a, sem_a)
            wait_gather(n_chunks - 2, rows_b, sem_b)
            scatter(n_chunks - 2, rows_b)
            wait_gather(n_chunks - 1, rows_a, sem_a)
            scatter(n_chunks - 1, rows_a)
        plsc.subcore_barrier()

        r0 = s * rows_per_tile
        pltpu.sync_copy(acc_sh.at[pl.ds(r0, rows_per_tile)],
                        out_hbm.at[c, pl.ds(r0, rows_per_tile)])

    return seg_sum(h, src, dst3)


_ROWS_BLK = 2000


def _mlp_body(x_ref, agg_ref, Wa_ref, ba_ref, Wb_ref, bb_ref, o_ref):
    z = x_ref[...] + agg_ref[0] + agg_ref[1]
    h1 = jnp.dot(z, Wa_ref[...], preferred_element_type=jnp.float32)
    h1 = jnp.maximum(h1 + ba_ref[...], 0.0)
    h2 = jnp.dot(h1, Wb_ref[...], preferred_element_type=jnp.float32)
    o_ref[...] = jnp.maximum(h2 + bb_ref[...], 0.0)


def _mlp(x, agg, Wa, ba, Wb, bb):
    N, D = x.shape
    H = Wa.shape[1]
    grid = (N // _ROWS_BLK,)
    return pl.pallas_call(
        _mlp_body,
        grid=grid,
        in_specs=[
            pl.BlockSpec((_ROWS_BLK, D), lambda i: (i, 0)),
            pl.BlockSpec((_NC, _ROWS_BLK, D), lambda i: (0, i, 0)),
            pl.BlockSpec((D, H), lambda i: (0, 0)),
            pl.BlockSpec((1, H), lambda i: (0, 0)),
            pl.BlockSpec((H, H), lambda i: (0, 0)),
            pl.BlockSpec((1, H), lambda i: (0, 0)),
        ],
        out_specs=pl.BlockSpec((_ROWS_BLK, H), lambda i: (i, 0)),
        out_shape=jax.ShapeDtypeStruct((N, H), jnp.float32),
    )(x, agg, Wa, ba, Wb, bb)


def kernel(x, edge_index, W1, b1, W2, b2, W3, b3, W4, b4):
    src = edge_index[0]
    dst = edge_index[1]
    b1r = b1.reshape(1, -1)
    b2r = b2.reshape(1, -1)
    b3r = b3.reshape(1, -1)
    b4r = b4.reshape(1, -1)
    agg1 = _segment_sum_sc(x, src, dst)
    h = _mlp(x, agg1, W1, b1r, W2, b2r)
    agg2 = _segment_sum_sc(h, src, dst)
    out = _mlp(h, agg2, W3, b3r, W4, b4r)
    return out
